# baseline (speedup 1.0000x reference)
import jax
import jax.numpy as jnp
from jax import lax
from jax.experimental import pallas as pl
from jax.experimental.pallas import tpu as pltpu
from jax.experimental.pallas import tpu_sc as plsc

N = 50000
E = 800000
D_IN = 9
H = 64
G = 512

NC = 2
NS = 16
C = 128
SUPER = 8

NB = 2000
NBLK = N // NB

NP = 50048

E_PAD2 = 800768
NR2 = E_PAD2 // C
NSUP2 = NR2 // SUPER

NSUP1R = 262
E_PAD1 = 3 * NSUP1R * SUPER * C
NR1 = E_PAD1 // C


def _conv_fm(tableT, srcR, dstR, attrR, w16, b16, zrow, conv1):
    mesh = plsc.VectorSubcoreMesh(core_axis_name="c", subcore_axis_name="s")
    out_rows = 32 if conv1 else H
    npass = 1 if conv1 else 2

    def body(tab_h, src_h, dst_h, attr_h, w_h, b_h, z_h, out_h,
             sv, dv, av, trow_v, acc_v, w_v, b_v, tsem, zsem, i0, i1):
        isems = [i0, i1]
        c = lax.axis_index("c")
        s = lax.axis_index("s")
        wid = c * NS + s
        if conv1:
            third = wid // 9
            active = third < 3
            start_sup = third * NSUP1R
            npairs = jnp.where(active, NSUP1R // 2, 0)
            nsup_t = NSUP1R
        else:
            active = wid >= 0
            start_sup = 0
            npairs = NSUP2 // 2
            nsup_t = NSUP2

        def stage(t, par):
            rb = (start_sup + t) * SUPER
            pb = par * SUPER
            pltpu.async_copy(src_h.at[pl.ds(rb, SUPER)],
                             sv.at[pl.ds(pb, SUPER)], isems[par])
            pltpu.async_copy(dst_h.at[pl.ds(rb, SUPER)],
                             dv.at[pl.ds(pb, SUPER)], isems[par])
            pltpu.async_copy(attr_h.at[pl.ds(rb, SUPER)],
                             av.at[pl.ds(pb, SUPER)], isems[par])

        def wait_stage(par):
            pb = par * SUPER
            pltpu.make_async_copy(src_h.at[pl.ds(0, SUPER)],
                                  sv.at[pl.ds(pb, SUPER)], isems[par]).wait()
            pltpu.make_async_copy(dst_h.at[pl.ds(0, SUPER)],
                                  dv.at[pl.ds(pb, SUPER)], isems[par]).wait()
            pltpu.make_async_copy(attr_h.at[pl.ds(0, SUPER)],
                                  av.at[pl.ds(pb, SUPER)], isems[par]).wait()

        for p in range(npass):
            if conv1:
                fid = wid % 9
                out_row = wid
            else:
                fid = p * 32 + wid
                out_row = fid
            pltpu.sync_copy(w_h.at[fid], w_v)
            pltpu.sync_copy(b_h.at[fid], b_v)
            wv = w_v[...]
            bv = b_v[...]
            tcopy = pltpu.async_copy(tab_h.at[fid], trow_v, tsem)
            zcopy = pltpu.async_copy(z_h, acc_v, zsem)

            @pl.when(active)
            def _prime():
                stage(0, 0)
            tcopy.wait()
            zcopy.wait()

            def sup2(t2, _):
                for par in (0, 1):
                    t = t2 * 2 + par

                    @pl.when(t + 1 < nsup_t)
                    def _next():
                        stage(t + 1, 1 - par)
                    wait_stage(par)

                    @plsc.parallel_loop(0, SUPER * (C // 16), unroll=4)
                    def _g(g):
                        row = par * SUPER + (g >> 3)
                        off = (g & 7) * 16
                        s16 = sv[row, pl.ds(off, 16)]
                        d16 = dv[row, pl.ds(off, 16)]
                        a16 = av[row, pl.ds(off, 16)]
                        vals = plsc.load_gather(trow_v, [s16])
                        m = jnp.maximum(vals + a16 * wv + bv, 0.0)
                        plsc.addupdate_scatter(acc_v, [d16], m)
                return 0
            lax.fori_loop(0, npairs, sup2, 0)
            pltpu.sync_copy(acc_v, out_h.at[out_row])

    return pl.kernel(
        body,
        out_type=jax.ShapeDtypeStruct((out_rows, NP), jnp.float32),
        mesh=mesh,
        compiler_params=pltpu.CompilerParams(use_tc_tiling_on_sc=False,
                                             needs_layout_passes=False),
        scratch_types=(
            [pltpu.VMEM((2 * SUPER, C), jnp.int32),
             pltpu.VMEM((2 * SUPER, C), jnp.int32),
             pltpu.VMEM((2 * SUPER, C), jnp.float32),
             pltpu.VMEM((NP,), jnp.float32),
             pltpu.VMEM((NP,), jnp.float32),
             pltpu.VMEM((16,), jnp.float32),
             pltpu.VMEM((16,), jnp.float32)]
            + [pltpu.SemaphoreType.DMA] * 4
        ),
    )(tableT, srcR, dstR, attrR, w16, b16, zrow)


def _da_body(h_ref, a_ref, w_ref, b_ref, h1_ref, s_ref, q_ref):
    h0 = h_ref[...] + a_ref[...]
    h1 = jnp.dot(h0, w_ref[...], preferred_element_type=jnp.float32) + b_ref[...]
    h1_ref[...] = h1
    s_ref[...] = jnp.sum(h1, axis=0, keepdims=True).reshape(1, 1, H)
    q_ref[...] = jnp.sum(h1 * h1, axis=0, keepdims=True).reshape(1, 1, H)


def _dense_a(h, agg, w, b):
    k = w.shape[0]
    return pl.pallas_call(
        _da_body,
        grid=(NBLK,),
        in_specs=[
            pl.BlockSpec((NB, k), lambda i: (i, 0)),
            pl.BlockSpec((NB, k), lambda i: (i, 0)),
            pl.BlockSpec((k, H), lambda i: (0, 0)),
            pl.BlockSpec((1, H), lambda i: (0, 0)),
        ],
        out_specs=[
            pl.BlockSpec((NB, H), lambda i: (i, 0)),
            pl.BlockSpec((1, 1, H), lambda i: (i, 0, 0)),
            pl.BlockSpec((1, 1, H), lambda i: (i, 0, 0)),
        ],
        out_shape=[
            jax.ShapeDtypeStruct((N, H), jnp.float32),
            jax.ShapeDtypeStruct((NBLK, 1, H), jnp.float32),
            jax.ShapeDtypeStruct((NBLK, 1, H), jnp.float32),
        ],
    )(h, agg, w, b)


def _db_body(h1_ref, s_ref, q_ref, g_ref, beta_ref, w_ref, b_ref, o_ref):
    mean = jnp.sum(s_ref[...], axis=0) / N
    ex2 = jnp.sum(q_ref[...], axis=0) / N
    var = ex2 - mean * mean
    inv = lax.rsqrt(var + 1e-5)
    hb = g_ref[...] * (h1_ref[...] - mean) * inv + beta_ref[...]
    hr = jnp.maximum(hb, 0.0)
    o = jnp.dot(hr, w_ref[...], preferred_element_type=jnp.float32) + b_ref[...]
    o_ref[...] = jnp.maximum(o, 0.0)


def _dense_b(h1, s, q, g, beta, w, b):
    return pl.pallas_call(
        _db_body,
        grid=(NBLK,),
        in_specs=[
            pl.BlockSpec((NB, H), lambda i: (i, 0)),
            pl.BlockSpec((NBLK, 1, H), lambda i: (0, 0, 0)),
            pl.BlockSpec((NBLK, 1, H), lambda i: (0, 0, 0)),
            pl.BlockSpec((1, H), lambda i: (0, 0)),
            pl.BlockSpec((1, H), lambda i: (0, 0)),
            pl.BlockSpec((H, H), lambda i: (0, 0)),
            pl.BlockSpec((1, H), lambda i: (0, 0)),
        ],
        out_specs=pl.BlockSpec((NB, H), lambda i: (i, 0)),
        out_shape=jax.ShapeDtypeStruct((N, H), jnp.float32),
    )(h1, s, q, g.reshape(1, H), beta.reshape(1, H), w, b)


def _pool_body(h_ref, bt_ref, wr1_ref, br1_ref, wr2_ref, br2_ref, o_ref,
               acc_ref, cnt_ref):
    i = pl.program_id(0)

    @pl.when(i == 0)
    def _init():
        acc_ref[...] = jnp.zeros_like(acc_ref)
        cnt_ref[...] = jnp.zeros_like(cnt_ref)

    bt = bt_ref[0]
    gid = lax.broadcasted_iota(jnp.int32, (G, NB), 0)
    oh = (gid == bt).astype(jnp.float32)
    acc_ref[...] += jnp.dot(oh, h_ref[...], preferred_element_type=jnp.float32)
    cnt_part = jnp.sum(oh, axis=1, keepdims=True)
    cnt_ref[...] += jnp.broadcast_to(cnt_part, (G, H))

    @pl.when(i == NBLK - 1)
    def _final():
        xg = acc_ref[...] / jnp.maximum(cnt_ref[...], 1.0)
        r = jnp.maximum(
            jnp.dot(xg, wr1_ref[...], preferred_element_type=jnp.float32)
            + br1_ref[...], 0.0)
        o = jnp.dot(r, wr2_ref[...], preferred_element_type=jnp.float32) + br2_ref[...]
        o_ref[...] = 1.0 / (1.0 + jnp.exp(-o))


def _pool_readout(h4, batch3, wr1, br1, wr2, br2):
    return pl.pallas_call(
        _pool_body,
        grid=(NBLK,),
        in_specs=[
            pl.BlockSpec((NB, H), lambda i: (i, 0)),
            pl.BlockSpec((1, 1, NB), lambda i: (i, 0, 0)),
            pl.BlockSpec((H, 32), lambda i: (0, 0)),
            pl.BlockSpec((1, 32), lambda i: (0, 0)),
            pl.BlockSpec((32, 1), lambda i: (0, 0)),
            pl.BlockSpec((1, 1), lambda i: (0, 0)),
        ],
        out_specs=pl.BlockSpec((G, 1), lambda i: (0, 0)),
        out_shape=jax.ShapeDtypeStruct((G, 1), jnp.float32),
        scratch_shapes=[
            pltpu.VMEM((G, H), jnp.float32),
            pltpu.VMEM((G, H), jnp.float32),
        ],
    )(h4, batch3, wr1, br1, wr2, br2)


def _edges_padded(src, dst, attr, e_pad):
    pad = e_pad - E
    nr = e_pad // C
    srcR = jnp.concatenate([src, jnp.zeros((pad,), jnp.int32)]).reshape(nr, C)
    dstR = jnp.concatenate([dst, jnp.full((pad,), N, jnp.int32)]).reshape(nr, C)
    attrR = jnp.concatenate([attr, jnp.zeros((pad,), jnp.float32)]).reshape(nr, C)
    return srcR, dstR, attrR


def kernel(x, edge_index, edge_attr, batch, params):
    p = params
    f32 = jnp.float32
    src = edge_index[0]
    dst = edge_index[1]
    attr = edge_attr[:, 0]
    zrow = jnp.zeros((NP,), f32)

    srcR1, dstR1, attrR1 = _edges_padded(src, dst, attr, E_PAD1)
    xT = jnp.pad(x.T, ((0, 0), (0, NP - N)))
    w16_1 = jnp.broadcast_to(p["We1"][0][:, None], (D_IN, 16))
    b16_1 = jnp.broadcast_to(p["be1"][:, None], (D_IN, 16))
    agg1P = _conv_fm(xT, srcR1, dstR1, attrR1, w16_1, b16_1, zrow,
                     conv1=True)
    agg1 = jnp.sum(agg1P[:27].reshape(3, 9, NP), axis=0)[:, :N].T

    x_pad = jnp.pad(x, ((0, 0), (0, 16 - D_IN)))
    agg1_pad = jnp.pad(agg1, ((0, 0), (0, 16 - D_IN)))
    w11p = jnp.pad(p["W11"], ((0, 16 - D_IN), (0, 0)))
    h1, s1, q1 = _dense_a(x_pad, agg1_pad, w11p, p["b11"].reshape(1, H))
    h2 = _dense_b(h1, s1, q1, p["g1"], p["beta1"], p["W12"],
                  p["b12"].reshape(1, H))

    srcR2, dstR2, attrR2 = _edges_padded(src, dst, attr, E_PAD2)
    table2T = jnp.pad(h2.T, ((0, 0), (0, NP - N)))
    w16_2 = jnp.broadcast_to(p["We2"][0][:, None], (H, 16))
    b16_2 = jnp.broadcast_to(p["be2"][:, None], (H, 16))
    aggT = _conv_fm(table2T, srcR2, dstR2, attrR2, w16_2, b16_2, zrow,
                    conv1=False)

    h3, s2, q2 = _dense_a(h2, aggT[:, :N].T, p["W21"], p["b21"].reshape(1, H))
    h4 = _dense_b(h3, s2, q2, p["g2"], p["beta2"], p["W22"],
                  p["b22"].reshape(1, H))

    batch3 = batch.reshape(NBLK, 1, NB)
    out = _pool_readout(h4, batch3, p["Wr1"], p["br1"].reshape(1, 32),
                        p["Wr2"], p["br2"].reshape(1, 1))
    return out


# speedup vs baseline: 1.0074x; 1.0074x over previous
import jax
import jax.numpy as jnp
from jax import lax
from jax.experimental import pallas as pl
from jax.experimental.pallas import tpu as pltpu
from jax.experimental.pallas import tpu_sc as plsc

N = 50000
E = 800000
D_IN = 9
H = 64
G = 512

NC = 2
NS = 16
C = 128
SUPER = 8

NB = 2000
NBLK = N // NB

NP = 50048

E_PAD2 = 800768
NR2 = E_PAD2 // C
NSUP2 = NR2 // SUPER

NSUP1R = 262
E_PAD1 = 3 * NSUP1R * SUPER * C
NR1 = E_PAD1 // C


def _conv_fm(tableT, srcR, dstR, attrR, w16, b16, zrow, conv1):
    mesh = plsc.VectorSubcoreMesh(core_axis_name="c", subcore_axis_name="s")
    out_rows = 32 if conv1 else H
    npass = 1 if conv1 else 2

    def body(tab_h, src_h, dst_h, attr_h, w_h, b_h, z_h, out_h,
             sv, dv, av, trow_v, acc_v, w_v, b_v, tsem, zsem, i0, i1):
        isems = [i0, i1]
        c = lax.axis_index("c")
        s = lax.axis_index("s")
        wid = c * NS + s
        if conv1:
            third = wid // 9
            active = third < 3
            start_sup = third * NSUP1R
            npairs = jnp.where(active, NSUP1R // 2, 0)
            nsup_t = NSUP1R
        else:
            active = wid >= 0
            start_sup = 0
            npairs = NSUP2 // 2
            nsup_t = NSUP2

        def stage(t, par):
            rb = (start_sup + t) * SUPER
            pb = par * SUPER
            pltpu.async_copy(src_h.at[pl.ds(rb, SUPER)],
                             sv.at[pl.ds(pb, SUPER)], isems[par])
            pltpu.async_copy(dst_h.at[pl.ds(rb, SUPER)],
                             dv.at[pl.ds(pb, SUPER)], isems[par])
            pltpu.async_copy(attr_h.at[pl.ds(rb, SUPER)],
                             av.at[pl.ds(pb, SUPER)], isems[par])

        def wait_stage(par):
            pb = par * SUPER
            pltpu.make_async_copy(src_h.at[pl.ds(0, SUPER)],
                                  sv.at[pl.ds(pb, SUPER)], isems[par]).wait()
            pltpu.make_async_copy(dst_h.at[pl.ds(0, SUPER)],
                                  dv.at[pl.ds(pb, SUPER)], isems[par]).wait()
            pltpu.make_async_copy(attr_h.at[pl.ds(0, SUPER)],
                                  av.at[pl.ds(pb, SUPER)], isems[par]).wait()

        for p in range(npass):
            if conv1:
                fid = wid % 9
                out_row = wid
            else:
                fid = p * 32 + wid
                out_row = fid
            pltpu.sync_copy(w_h.at[fid], w_v)
            pltpu.sync_copy(b_h.at[fid], b_v)
            wv = w_v[...]
            bv = b_v[...]
            tcopy = pltpu.async_copy(tab_h.at[fid], trow_v, tsem)
            zcopy = pltpu.async_copy(z_h, acc_v, zsem)

            @pl.when(active)
            def _prime():
                stage(0, 0)
            tcopy.wait()
            zcopy.wait()

            def sup2(t2, _):
                for par in (0, 1):
                    t = t2 * 2 + par

                    @pl.when(t + 1 < nsup_t)
                    def _next():
                        stage(t + 1, 1 - par)
                    wait_stage(par)

                    @plsc.parallel_loop(0, SUPER * (C // 16), unroll=2)
                    def _g(g):
                        row = par * SUPER + (g >> 3)
                        off = (g & 7) * 16
                        s16 = sv[row, pl.ds(off, 16)]
                        d16 = dv[row, pl.ds(off, 16)]
                        a16 = av[row, pl.ds(off, 16)]
                        vals = plsc.load_gather(trow_v, [s16])
                        m = jnp.maximum(vals + a16 * wv + bv, 0.0)
                        plsc.addupdate_scatter(acc_v, [d16], m)
                return 0
            lax.fori_loop(0, npairs, sup2, 0)
            pltpu.sync_copy(acc_v, out_h.at[out_row])

    return pl.kernel(
        body,
        out_type=jax.ShapeDtypeStruct((out_rows, NP), jnp.float32),
        mesh=mesh,
        compiler_params=pltpu.CompilerParams(use_tc_tiling_on_sc=False,
                                             needs_layout_passes=False),
        scratch_types=(
            [pltpu.VMEM((2 * SUPER, C), jnp.int32),
             pltpu.VMEM((2 * SUPER, C), jnp.int32),
             pltpu.VMEM((2 * SUPER, C), jnp.float32),
             pltpu.VMEM((NP,), jnp.float32),
             pltpu.VMEM((NP,), jnp.float32),
             pltpu.VMEM((16,), jnp.float32),
             pltpu.VMEM((16,), jnp.float32)]
            + [pltpu.SemaphoreType.DMA] * 4
        ),
    )(tableT, srcR, dstR, attrR, w16, b16, zrow)


def _da_body(h_ref, a_ref, w_ref, b_ref, h1_ref, s_ref, q_ref):
    h0 = h_ref[...] + a_ref[...]
    h1 = jnp.dot(h0, w_ref[...], preferred_element_type=jnp.float32) + b_ref[...]
    h1_ref[...] = h1
    s_ref[...] = jnp.sum(h1, axis=0, keepdims=True).reshape(1, 1, H)
    q_ref[...] = jnp.sum(h1 * h1, axis=0, keepdims=True).reshape(1, 1, H)


def _dense_a(h, agg, w, b):
    k = w.shape[0]
    return pl.pallas_call(
        _da_body,
        grid=(NBLK,),
        in_specs=[
            pl.BlockSpec((NB, k), lambda i: (i, 0)),
            pl.BlockSpec((NB, k), lambda i: (i, 0)),
            pl.BlockSpec((k, H), lambda i: (0, 0)),
            pl.BlockSpec((1, H), lambda i: (0, 0)),
        ],
        out_specs=[
            pl.BlockSpec((NB, H), lambda i: (i, 0)),
            pl.BlockSpec((1, 1, H), lambda i: (i, 0, 0)),
            pl.BlockSpec((1, 1, H), lambda i: (i, 0, 0)),
        ],
        out_shape=[
            jax.ShapeDtypeStruct((N, H), jnp.float32),
            jax.ShapeDtypeStruct((NBLK, 1, H), jnp.float32),
            jax.ShapeDtypeStruct((NBLK, 1, H), jnp.float32),
        ],
    )(h, agg, w, b)


def _db_body(h1_ref, s_ref, q_ref, g_ref, beta_ref, w_ref, b_ref, o_ref):
    mean = jnp.sum(s_ref[...], axis=0) / N
    ex2 = jnp.sum(q_ref[...], axis=0) / N
    var = ex2 - mean * mean
    inv = lax.rsqrt(var + 1e-5)
    hb = g_ref[...] * (h1_ref[...] - mean) * inv + beta_ref[...]
    hr = jnp.maximum(hb, 0.0)
    o = jnp.dot(hr, w_ref[...], preferred_element_type=jnp.float32) + b_ref[...]
    o_ref[...] = jnp.maximum(o, 0.0)


def _dense_b(h1, s, q, g, beta, w, b):
    return pl.pallas_call(
        _db_body,
        grid=(NBLK,),
        in_specs=[
            pl.BlockSpec((NB, H), lambda i: (i, 0)),
            pl.BlockSpec((NBLK, 1, H), lambda i: (0, 0, 0)),
            pl.BlockSpec((NBLK, 1, H), lambda i: (0, 0, 0)),
            pl.BlockSpec((1, H), lambda i: (0, 0)),
            pl.BlockSpec((1, H), lambda i: (0, 0)),
            pl.BlockSpec((H, H), lambda i: (0, 0)),
            pl.BlockSpec((1, H), lambda i: (0, 0)),
        ],
        out_specs=pl.BlockSpec((NB, H), lambda i: (i, 0)),
        out_shape=jax.ShapeDtypeStruct((N, H), jnp.float32),
    )(h1, s, q, g.reshape(1, H), beta.reshape(1, H), w, b)


def _pool_body(h_ref, bt_ref, wr1_ref, br1_ref, wr2_ref, br2_ref, o_ref,
               acc_ref, cnt_ref):
    i = pl.program_id(0)

    @pl.when(i == 0)
    def _init():
        acc_ref[...] = jnp.zeros_like(acc_ref)
        cnt_ref[...] = jnp.zeros_like(cnt_ref)

    bt = bt_ref[0]
    gid = lax.broadcasted_iota(jnp.int32, (G, NB), 0)
    oh = (gid == bt).astype(jnp.float32)
    acc_ref[...] += jnp.dot(oh, h_ref[...], preferred_element_type=jnp.float32)
    cnt_part = jnp.sum(oh, axis=1, keepdims=True)
    cnt_ref[...] += jnp.broadcast_to(cnt_part, (G, H))

    @pl.when(i == NBLK - 1)
    def _final():
        xg = acc_ref[...] / jnp.maximum(cnt_ref[...], 1.0)
        r = jnp.maximum(
            jnp.dot(xg, wr1_ref[...], preferred_element_type=jnp.float32)
            + br1_ref[...], 0.0)
        o = jnp.dot(r, wr2_ref[...], preferred_element_type=jnp.float32) + br2_ref[...]
        o_ref[...] = 1.0 / (1.0 + jnp.exp(-o))


def _pool_readout(h4, batch3, wr1, br1, wr2, br2):
    return pl.pallas_call(
        _pool_body,
        grid=(NBLK,),
        in_specs=[
            pl.BlockSpec((NB, H), lambda i: (i, 0)),
            pl.BlockSpec((1, 1, NB), lambda i: (i, 0, 0)),
            pl.BlockSpec((H, 32), lambda i: (0, 0)),
            pl.BlockSpec((1, 32), lambda i: (0, 0)),
            pl.BlockSpec((32, 1), lambda i: (0, 0)),
            pl.BlockSpec((1, 1), lambda i: (0, 0)),
        ],
        out_specs=pl.BlockSpec((G, 1), lambda i: (0, 0)),
        out_shape=jax.ShapeDtypeStruct((G, 1), jnp.float32),
        scratch_shapes=[
            pltpu.VMEM((G, H), jnp.float32),
            pltpu.VMEM((G, H), jnp.float32),
        ],
    )(h4, batch3, wr1, br1, wr2, br2)


def _edges_padded(src, dst, attr, e_pad):
    pad = e_pad - E
    nr = e_pad // C
    srcR = jnp.concatenate([src, jnp.zeros((pad,), jnp.int32)]).reshape(nr, C)
    dstR = jnp.concatenate([dst, jnp.full((pad,), N, jnp.int32)]).reshape(nr, C)
    attrR = jnp.concatenate([attr, jnp.zeros((pad,), jnp.float32)]).reshape(nr, C)
    return srcR, dstR, attrR


def kernel(x, edge_index, edge_attr, batch, params):
    p = params
    f32 = jnp.float32
    src = edge_index[0]
    dst = edge_index[1]
    attr = edge_attr[:, 0]
    zrow = jnp.zeros((NP,), f32)

    srcR1, dstR1, attrR1 = _edges_padded(src, dst, attr, E_PAD1)
    xT = jnp.pad(x.T, ((0, 0), (0, NP - N)))
    w16_1 = jnp.broadcast_to(p["We1"][0][:, None], (D_IN, 16))
    b16_1 = jnp.broadcast_to(p["be1"][:, None], (D_IN, 16))
    agg1P = _conv_fm(xT, srcR1, dstR1, attrR1, w16_1, b16_1, zrow,
                     conv1=True)
    agg1 = jnp.sum(agg1P[:27].reshape(3, 9, NP), axis=0)[:, :N].T

    x_pad = jnp.pad(x, ((0, 0), (0, 16 - D_IN)))
    agg1_pad = jnp.pad(agg1, ((0, 0), (0, 16 - D_IN)))
    w11p = jnp.pad(p["W11"], ((0, 16 - D_IN), (0, 0)))
    h1, s1, q1 = _dense_a(x_pad, agg1_pad, w11p, p["b11"].reshape(1, H))
    h2 = _dense_b(h1, s1, q1, p["g1"], p["beta1"], p["W12"],
                  p["b12"].reshape(1, H))

    srcR2, dstR2, attrR2 = _edges_padded(src, dst, attr, E_PAD2)
    table2T = jnp.pad(h2.T, ((0, 0), (0, NP - N)))
    w16_2 = jnp.broadcast_to(p["We2"][0][:, None], (H, 16))
    b16_2 = jnp.broadcast_to(p["be2"][:, None], (H, 16))
    aggT = _conv_fm(table2T, srcR2, dstR2, attrR2, w16_2, b16_2, zrow,
                    conv1=False)

    h3, s2, q2 = _dense_a(h2, aggT[:, :N].T, p["W21"], p["b21"].reshape(1, H))
    h4 = _dense_b(h3, s2, q2, p["g2"], p["beta2"], p["W22"],
                  p["b22"].reshape(1, H))

    batch3 = batch.reshape(NBLK, 1, NB)
    out = _pool_readout(h4, batch3, p["Wr1"], p["br1"].reshape(1, 32),
                        p["Wr2"], p["br2"].reshape(1, 1))
    return out


# speedup vs baseline: 1.0847x; 1.0768x over previous
import jax
import jax.numpy as jnp
from jax import lax
from jax.experimental import pallas as pl
from jax.experimental.pallas import tpu as pltpu
from jax.experimental.pallas import tpu_sc as plsc

N = 50000
E = 800000
D_IN = 9
H = 64
G = 512

NC = 2
NS = 16
C = 128
SUPER = 8

NB = 2000
NBLK = N // NB

NP = 50048

E_PAD2 = 800768
NR2 = E_PAD2 // C
NSUP2 = NR2 // SUPER

NSUP1R = 262
E_PAD1 = 3 * NSUP1R * SUPER * C
NR1 = E_PAD1 // C


def _conv_fm(tableT, sdR, attrR, w16, b16, zrow, conv1):
    mesh = plsc.VectorSubcoreMesh(core_axis_name="c", subcore_axis_name="s")
    out_rows = 32 if conv1 else H
    npass = 1 if conv1 else 2

    def body(tab_h, sd_h, attr_h, w_h, b_h, z_h, out_h,
             sdv, av, trow_v, acc_v, w_v, b_v, tsem, zsem, i0, i1):
        isems = [i0, i1]
        c = lax.axis_index("c")
        s = lax.axis_index("s")
        wid = c * NS + s
        if conv1:
            third = wid // 9
            active = third < 3
            start_sup = third * NSUP1R
            npairs = jnp.where(active, NSUP1R // 2, 0)
            nsup_t = NSUP1R
        else:
            active = wid >= 0
            start_sup = 0
            npairs = NSUP2 // 2
            nsup_t = NSUP2

        def stage(t, par):
            rb = (start_sup + t) * SUPER
            pb = par * SUPER
            pltpu.async_copy(sd_h.at[pl.ds(rb, SUPER)],
                             sdv.at[pl.ds(pb, SUPER)], isems[par])
            pltpu.async_copy(attr_h.at[pl.ds(rb, SUPER)],
                             av.at[pl.ds(pb, SUPER)], isems[par])

        def wait_stage(par):
            pb = par * SUPER
            pltpu.make_async_copy(sd_h.at[pl.ds(0, SUPER)],
                                  sdv.at[pl.ds(pb, SUPER)], isems[par]).wait()
            pltpu.make_async_copy(attr_h.at[pl.ds(0, SUPER)],
                                  av.at[pl.ds(pb, SUPER)], isems[par]).wait()

        for p in range(npass):
            if conv1:
                fid = wid % 9
                out_row = wid
            else:
                fid = p * 32 + wid
                out_row = fid
            pltpu.sync_copy(w_h.at[fid], w_v)
            pltpu.sync_copy(b_h.at[fid], b_v)
            wv = w_v[...]
            bv = b_v[...]
            tcopy = pltpu.async_copy(tab_h.at[fid], trow_v, tsem)
            zcopy = pltpu.async_copy(z_h, acc_v, zsem)

            @pl.when(active)
            def _prime():
                stage(0, 0)
            tcopy.wait()
            zcopy.wait()

            def sup2(t2, _):
                for par in (0, 1):
                    t = t2 * 2 + par

                    @pl.when(t + 1 < nsup_t)
                    def _next():
                        stage(t + 1, 1 - par)
                    wait_stage(par)

                    @plsc.parallel_loop(0, SUPER * (C // 16), unroll=2)
                    def _g(g):
                        row = par * SUPER + (g >> 3)
                        off = (g & 7) * 16
                        sd16 = sdv[row, pl.ds(off, 16)]
                        a16 = av[row, pl.ds(off, 16)]
                        s16 = sd16 & 0xFFFF
                        d16 = lax.shift_right_logical(sd16, 16)
                        vals = plsc.load_gather(trow_v, [s16])
                        m = jnp.maximum(vals + a16 * wv + bv, 0.0)
                        plsc.addupdate_scatter(acc_v, [d16], m)
                return 0
            lax.fori_loop(0, npairs, sup2, 0)
            pltpu.sync_copy(acc_v, out_h.at[out_row])

    return pl.kernel(
        body,
        out_type=jax.ShapeDtypeStruct((out_rows, NP), jnp.float32),
        mesh=mesh,
        compiler_params=pltpu.CompilerParams(use_tc_tiling_on_sc=False,
                                             needs_layout_passes=False),
        scratch_types=(
            [pltpu.VMEM((2 * SUPER, C), jnp.int32),
             pltpu.VMEM((2 * SUPER, C), jnp.float32),
             pltpu.VMEM((NP,), jnp.float32),
             pltpu.VMEM((NP,), jnp.float32),
             pltpu.VMEM((16,), jnp.float32),
             pltpu.VMEM((16,), jnp.float32)]
            + [pltpu.SemaphoreType.DMA] * 4
        ),
    )(tableT, sdR, attrR, w16, b16, zrow)


def _da_body(h_ref, a_ref, w_ref, b_ref, h1_ref, s_ref, q_ref):
    h0 = h_ref[...] + a_ref[...]
    h1 = jnp.dot(h0, w_ref[...], preferred_element_type=jnp.float32) + b_ref[...]
    h1_ref[...] = h1
    s_ref[...] = jnp.sum(h1, axis=0, keepdims=True).reshape(1, 1, H)
    q_ref[...] = jnp.sum(h1 * h1, axis=0, keepdims=True).reshape(1, 1, H)


def _dense_a(h, agg, w, b):
    k = w.shape[0]
    return pl.pallas_call(
        _da_body,
        grid=(NBLK,),
        in_specs=[
            pl.BlockSpec((NB, k), lambda i: (i, 0)),
            pl.BlockSpec((NB, k), lambda i: (i, 0)),
            pl.BlockSpec((k, H), lambda i: (0, 0)),
            pl.BlockSpec((1, H), lambda i: (0, 0)),
        ],
        out_specs=[
            pl.BlockSpec((NB, H), lambda i: (i, 0)),
            pl.BlockSpec((1, 1, H), lambda i: (i, 0, 0)),
            pl.BlockSpec((1, 1, H), lambda i: (i, 0, 0)),
        ],
        out_shape=[
            jax.ShapeDtypeStruct((N, H), jnp.float32),
            jax.ShapeDtypeStruct((NBLK, 1, H), jnp.float32),
            jax.ShapeDtypeStruct((NBLK, 1, H), jnp.float32),
        ],
    )(h, agg, w, b)


def _db_body(h1_ref, s_ref, q_ref, g_ref, beta_ref, w_ref, b_ref, o_ref):
    mean = jnp.sum(s_ref[...], axis=0) / N
    ex2 = jnp.sum(q_ref[...], axis=0) / N
    var = ex2 - mean * mean
    inv = lax.rsqrt(var + 1e-5)
    hb = g_ref[...] * (h1_ref[...] - mean) * inv + beta_ref[...]
    hr = jnp.maximum(hb, 0.0)
    o = jnp.dot(hr, w_ref[...], preferred_element_type=jnp.float32) + b_ref[...]
    o_ref[...] = jnp.maximum(o, 0.0)


def _dense_b(h1, s, q, g, beta, w, b):
    return pl.pallas_call(
        _db_body,
        grid=(NBLK,),
        in_specs=[
            pl.BlockSpec((NB, H), lambda i: (i, 0)),
            pl.BlockSpec((NBLK, 1, H), lambda i: (0, 0, 0)),
            pl.BlockSpec((NBLK, 1, H), lambda i: (0, 0, 0)),
            pl.BlockSpec((1, H), lambda i: (0, 0)),
            pl.BlockSpec((1, H), lambda i: (0, 0)),
            pl.BlockSpec((H, H), lambda i: (0, 0)),
            pl.BlockSpec((1, H), lambda i: (0, 0)),
        ],
        out_specs=pl.BlockSpec((NB, H), lambda i: (i, 0)),
        out_shape=jax.ShapeDtypeStruct((N, H), jnp.float32),
    )(h1, s, q, g.reshape(1, H), beta.reshape(1, H), w, b)


def _pool_body(h_ref, bt_ref, wr1_ref, br1_ref, wr2_ref, br2_ref, o_ref,
               acc_ref, cnt_ref):
    i = pl.program_id(0)

    @pl.when(i == 0)
    def _init():
        acc_ref[...] = jnp.zeros_like(acc_ref)
        cnt_ref[...] = jnp.zeros_like(cnt_ref)

    bt = bt_ref[0]
    gid = lax.broadcasted_iota(jnp.int32, (G, NB), 0)
    oh = (gid == bt).astype(jnp.float32)
    acc_ref[...] += jnp.dot(oh, h_ref[...], preferred_element_type=jnp.float32)
    cnt_part = jnp.sum(oh, axis=1, keepdims=True)
    cnt_ref[...] += jnp.broadcast_to(cnt_part, (G, H))

    @pl.when(i == NBLK - 1)
    def _final():
        xg = acc_ref[...] / jnp.maximum(cnt_ref[...], 1.0)
        r = jnp.maximum(
            jnp.dot(xg, wr1_ref[...], preferred_element_type=jnp.float32)
            + br1_ref[...], 0.0)
        o = jnp.dot(r, wr2_ref[...], preferred_element_type=jnp.float32) + br2_ref[...]
        o_ref[...] = 1.0 / (1.0 + jnp.exp(-o))


def _pool_readout(h4, batch3, wr1, br1, wr2, br2):
    return pl.pallas_call(
        _pool_body,
        grid=(NBLK,),
        in_specs=[
            pl.BlockSpec((NB, H), lambda i: (i, 0)),
            pl.BlockSpec((1, 1, NB), lambda i: (i, 0, 0)),
            pl.BlockSpec((H, 32), lambda i: (0, 0)),
            pl.BlockSpec((1, 32), lambda i: (0, 0)),
            pl.BlockSpec((32, 1), lambda i: (0, 0)),
            pl.BlockSpec((1, 1), lambda i: (0, 0)),
        ],
        out_specs=pl.BlockSpec((G, 1), lambda i: (0, 0)),
        out_shape=jax.ShapeDtypeStruct((G, 1), jnp.float32),
        scratch_shapes=[
            pltpu.VMEM((G, H), jnp.float32),
            pltpu.VMEM((G, H), jnp.float32),
        ],
    )(h4, batch3, wr1, br1, wr2, br2)


def _edges_padded(sd, attr, e_pad):
    pad = e_pad - E
    nr = e_pad // C
    sdR = jnp.concatenate([sd, jnp.full((pad,), N << 16, jnp.int32)]).reshape(nr, C)
    attrR = jnp.concatenate([attr, jnp.zeros((pad,), jnp.float32)]).reshape(nr, C)
    return sdR, attrR


def kernel(x, edge_index, edge_attr, batch, params):
    p = params
    f32 = jnp.float32
    src = edge_index[0]
    dst = edge_index[1]
    attr = edge_attr[:, 0]
    sd = src | (dst << 16)
    zrow = jnp.zeros((NP,), f32)

    sdR1, attrR1 = _edges_padded(sd, attr, E_PAD1)
    xT = jnp.pad(x.T, ((0, 0), (0, NP - N)))
    w16_1 = jnp.broadcast_to(p["We1"][0][:, None], (D_IN, 16))
    b16_1 = jnp.broadcast_to(p["be1"][:, None], (D_IN, 16))
    agg1P = _conv_fm(xT, sdR1, attrR1, w16_1, b16_1, zrow,
                     conv1=True)
    agg1 = jnp.sum(agg1P[:27].reshape(3, 9, NP), axis=0)[:, :N].T

    x_pad = jnp.pad(x, ((0, 0), (0, 16 - D_IN)))
    agg1_pad = jnp.pad(agg1, ((0, 0), (0, 16 - D_IN)))
    w11p = jnp.pad(p["W11"], ((0, 16 - D_IN), (0, 0)))
    h1, s1, q1 = _dense_a(x_pad, agg1_pad, w11p, p["b11"].reshape(1, H))
    h2 = _dense_b(h1, s1, q1, p["g1"], p["beta1"], p["W12"],
                  p["b12"].reshape(1, H))

    sdR2, attrR2 = _edges_padded(sd, attr, E_PAD2)
    table2T = jnp.pad(h2.T, ((0, 0), (0, NP - N)))
    w16_2 = jnp.broadcast_to(p["We2"][0][:, None], (H, 16))
    b16_2 = jnp.broadcast_to(p["be2"][:, None], (H, 16))
    aggT = _conv_fm(table2T, sdR2, attrR2, w16_2, b16_2, zrow,
                    conv1=False)

    h3, s2, q2 = _dense_a(h2, aggT[:, :N].T, p["W21"], p["b21"].reshape(1, H))
    h4 = _dense_b(h3, s2, q2, p["g2"], p["beta2"], p["W22"],
                  p["b22"].reshape(1, H))

    batch3 = batch.reshape(NBLK, 1, NB)
    out = _pool_readout(h4, batch3, p["Wr1"], p["br1"].reshape(1, 32),
                        p["Wr2"], p["br2"].reshape(1, 1))
    return out


# speedup vs baseline: 1.0851x; 1.0003x over previous
import jax
import jax.numpy as jnp
from jax import lax
from jax.experimental import pallas as pl
from jax.experimental.pallas import tpu as pltpu
from jax.experimental.pallas import tpu_sc as plsc

N = 50000
E = 800000
D_IN = 9
H = 64
G = 512

NC = 2
NS = 16
C = 128
SUPER = 8

NB = 2000
NBLK = N // NB

NP = 50048

E_PAD2 = 800768
NR2 = E_PAD2 // C
NSUP2 = NR2 // SUPER

NSUP1R = 262
E_PAD1 = 3 * NSUP1R * SUPER * C
NR1 = E_PAD1 // C


def _conv_fm(tableT, sdR, attrR, w16, b16, zrow, conv1):
    mesh = plsc.VectorSubcoreMesh(core_axis_name="c", subcore_axis_name="s")
    out_rows = 32 if conv1 else H
    npass = 1 if conv1 else 2

    def body(tab_h, sd_h, attr_h, w_h, b_h, z_h, out_h,
             sdv, av, trow_v, acc_v, w_v, b_v, tsem, zsem, i0, i1):
        isems = [i0, i1]
        c = lax.axis_index("c")
        s = lax.axis_index("s")
        wid = c * NS + s
        if conv1:
            third = wid // 9
            active = third < 3
            start_sup = third * NSUP1R
            npairs = jnp.where(active, NSUP1R // 2, 0)
            nsup_t = NSUP1R
        else:
            active = wid >= 0
            start_sup = 0
            npairs = NSUP2 // 2
            nsup_t = NSUP2

        def stage(t, par):
            rb = (start_sup + t) * SUPER
            pb = par * SUPER
            pltpu.async_copy(sd_h.at[pl.ds(rb, SUPER)],
                             sdv.at[pl.ds(pb, SUPER)], isems[par])
            pltpu.async_copy(attr_h.at[pl.ds(rb, SUPER)],
                             av.at[pl.ds(pb, SUPER)], isems[par])

        def wait_stage(par):
            pb = par * SUPER
            pltpu.make_async_copy(sd_h.at[pl.ds(0, SUPER)],
                                  sdv.at[pl.ds(pb, SUPER)], isems[par]).wait()
            pltpu.make_async_copy(attr_h.at[pl.ds(0, SUPER)],
                                  av.at[pl.ds(pb, SUPER)], isems[par]).wait()

        for p in range(npass):
            if conv1:
                fid = wid % 9
                out_row = wid
            else:
                fid = p * 32 + wid
                out_row = fid
            pltpu.sync_copy(w_h.at[fid], w_v)
            pltpu.sync_copy(b_h.at[fid], b_v)
            wv = w_v[...]
            bv = b_v[...]
            tcopy = pltpu.async_copy(tab_h.at[fid], trow_v, tsem)
            zcopy = pltpu.async_copy(z_h, acc_v, zsem)

            @pl.when(active)
            def _prime():
                stage(0, 0)
            tcopy.wait()
            zcopy.wait()

            def sup2(t2, _):
                for par in (0, 1):
                    t = t2 * 2 + par

                    @pl.when(t + 1 < nsup_t)
                    def _next():
                        stage(t + 1, 1 - par)
                    wait_stage(par)

                    @plsc.parallel_loop(0, SUPER * (C // 16), unroll=2)
                    def _g(g):
                        row = par * SUPER + (g >> 3)
                        off = (g & 7) * 16
                        sd16 = sdv[row, pl.ds(off, 16)]
                        a16 = av[row, pl.ds(off, 16)]
                        s16 = sd16 & 0xFFFF
                        d16 = lax.shift_right_logical(sd16, 16)
                        vals = plsc.load_gather(trow_v, [s16])
                        m = jnp.maximum(vals + a16 * wv + bv, 0.0)
                        plsc.addupdate_scatter(acc_v, [d16], m)
                return 0
            lax.fori_loop(0, npairs, sup2, 0)
            pltpu.sync_copy(acc_v, out_h.at[out_row])

    return pl.kernel(
        body,
        out_type=jax.ShapeDtypeStruct((out_rows, NP), jnp.float32),
        mesh=mesh,
        compiler_params=pltpu.CompilerParams(use_tc_tiling_on_sc=False,
                                             needs_layout_passes=False),
        scratch_types=(
            [pltpu.VMEM((2 * SUPER, C), jnp.int32),
             pltpu.VMEM((2 * SUPER, C), jnp.float32),
             pltpu.VMEM((NP,), jnp.float32),
             pltpu.VMEM((NP,), jnp.float32),
             pltpu.VMEM((16,), jnp.float32),
             pltpu.VMEM((16,), jnp.float32)]
            + [pltpu.SemaphoreType.DMA] * 4
        ),
    )(tableT, sdR, attrR, w16, b16, zrow)


def _da_body(h_ref, a_ref, w_ref, b_ref, h1_ref, s_ref, q_ref):
    if a_ref.shape[0] == h_ref.shape[0]:
        h0 = h_ref[...] + a_ref[...]
    else:
        h0 = h_ref[...] + a_ref[...].T
    h1 = jnp.dot(h0, w_ref[...], preferred_element_type=jnp.float32) + b_ref[...]
    h1_ref[...] = h1
    s_ref[...] = jnp.sum(h1, axis=0, keepdims=True).reshape(1, 1, H)
    q_ref[...] = jnp.sum(h1 * h1, axis=0, keepdims=True).reshape(1, 1, H)


def _dense_a(h, agg, w, b, agg_t=False):
    k = w.shape[0]
    aspec = (pl.BlockSpec((k, NB), lambda i: (0, i)) if agg_t
             else pl.BlockSpec((NB, k), lambda i: (i, 0)))
    return pl.pallas_call(
        _da_body,
        grid=(NBLK,),
        in_specs=[
            pl.BlockSpec((NB, k), lambda i: (i, 0)),
            aspec,
            pl.BlockSpec((k, H), lambda i: (0, 0)),
            pl.BlockSpec((1, H), lambda i: (0, 0)),
        ],
        out_specs=[
            pl.BlockSpec((NB, H), lambda i: (i, 0)),
            pl.BlockSpec((1, 1, H), lambda i: (i, 0, 0)),
            pl.BlockSpec((1, 1, H), lambda i: (i, 0, 0)),
        ],
        out_shape=[
            jax.ShapeDtypeStruct((N, H), jnp.float32),
            jax.ShapeDtypeStruct((NBLK, 1, H), jnp.float32),
            jax.ShapeDtypeStruct((NBLK, 1, H), jnp.float32),
        ],
    )(h, agg, w, b)


def _db_body(h1_ref, s_ref, q_ref, g_ref, beta_ref, w_ref, b_ref, *out_refs):
    mean = jnp.sum(s_ref[...], axis=0) / N
    ex2 = jnp.sum(q_ref[...], axis=0) / N
    var = ex2 - mean * mean
    inv = lax.rsqrt(var + 1e-5)
    hb = g_ref[...] * (h1_ref[...] - mean) * inv + beta_ref[...]
    hr = jnp.maximum(hb, 0.0)
    o = jnp.dot(hr, w_ref[...], preferred_element_type=jnp.float32) + b_ref[...]
    o = jnp.maximum(o, 0.0)
    out_refs[0][...] = o
    if len(out_refs) > 1:
        out_refs[1][...] = o.T


def _dense_b(h1, s, q, g, beta, w, b, emit_t=False):
    out_specs = [pl.BlockSpec((NB, H), lambda i: (i, 0))]
    out_shape = [jax.ShapeDtypeStruct((N, H), jnp.float32)]
    if emit_t:
        out_specs.append(pl.BlockSpec((H, NB), lambda i: (0, i)))
        out_shape.append(jax.ShapeDtypeStruct((H, NP), jnp.float32))
    res = pl.pallas_call(
        _db_body,
        grid=(NBLK,),
        in_specs=[
            pl.BlockSpec((NB, H), lambda i: (i, 0)),
            pl.BlockSpec((NBLK, 1, H), lambda i: (0, 0, 0)),
            pl.BlockSpec((NBLK, 1, H), lambda i: (0, 0, 0)),
            pl.BlockSpec((1, H), lambda i: (0, 0)),
            pl.BlockSpec((1, H), lambda i: (0, 0)),
            pl.BlockSpec((H, H), lambda i: (0, 0)),
            pl.BlockSpec((1, H), lambda i: (0, 0)),
        ],
        out_specs=out_specs,
        out_shape=out_shape,
    )(h1, s, q, g.reshape(1, H), beta.reshape(1, H), w, b)
    return res if emit_t else res[0]


def _pool_body(h_ref, bt_ref, wr1_ref, br1_ref, wr2_ref, br2_ref, o_ref,
               acc_ref, cnt_ref):
    i = pl.program_id(0)

    @pl.when(i == 0)
    def _init():
        acc_ref[...] = jnp.zeros_like(acc_ref)
        cnt_ref[...] = jnp.zeros_like(cnt_ref)

    bt = bt_ref[0]
    gid = lax.broadcasted_iota(jnp.int32, (G, NB), 0)
    oh = (gid == bt).astype(jnp.float32)
    acc_ref[...] += jnp.dot(oh, h_ref[...], preferred_element_type=jnp.float32)
    cnt_part = jnp.sum(oh, axis=1, keepdims=True)
    cnt_ref[...] += jnp.broadcast_to(cnt_part, (G, H))

    @pl.when(i == NBLK - 1)
    def _final():
        xg = acc_ref[...] / jnp.maximum(cnt_ref[...], 1.0)
        r = jnp.maximum(
            jnp.dot(xg, wr1_ref[...], preferred_element_type=jnp.float32)
            + br1_ref[...], 0.0)
        o = jnp.dot(r, wr2_ref[...], preferred_element_type=jnp.float32) + br2_ref[...]
        o_ref[...] = 1.0 / (1.0 + jnp.exp(-o))


def _pool_readout(h4, batch3, wr1, br1, wr2, br2):
    return pl.pallas_call(
        _pool_body,
        grid=(NBLK,),
        in_specs=[
            pl.BlockSpec((NB, H), lambda i: (i, 0)),
            pl.BlockSpec((1, 1, NB), lambda i: (i, 0, 0)),
            pl.BlockSpec((H, 32), lambda i: (0, 0)),
            pl.BlockSpec((1, 32), lambda i: (0, 0)),
            pl.BlockSpec((32, 1), lambda i: (0, 0)),
            pl.BlockSpec((1, 1), lambda i: (0, 0)),
        ],
        out_specs=pl.BlockSpec((G, 1), lambda i: (0, 0)),
        out_shape=jax.ShapeDtypeStruct((G, 1), jnp.float32),
        scratch_shapes=[
            pltpu.VMEM((G, H), jnp.float32),
            pltpu.VMEM((G, H), jnp.float32),
        ],
    )(h4, batch3, wr1, br1, wr2, br2)


def _edges_padded(sd, attr, e_pad):
    pad = e_pad - E
    nr = e_pad // C
    sdR = jnp.concatenate([sd, jnp.full((pad,), N << 16, jnp.int32)]).reshape(nr, C)
    attrR = jnp.concatenate([attr, jnp.zeros((pad,), jnp.float32)]).reshape(nr, C)
    return sdR, attrR


def kernel(x, edge_index, edge_attr, batch, params):
    p = params
    f32 = jnp.float32
    src = edge_index[0]
    dst = edge_index[1]
    attr = edge_attr[:, 0]
    sd = src | (dst << 16)
    zrow = jnp.zeros((NP,), f32)

    sdR1, attrR1 = _edges_padded(sd, attr, E_PAD1)
    xT = jnp.pad(x.T, ((0, 0), (0, NP - N)))
    w16_1 = jnp.broadcast_to(p["We1"][0][:, None], (D_IN, 16))
    b16_1 = jnp.broadcast_to(p["be1"][:, None], (D_IN, 16))
    agg1P = _conv_fm(xT, sdR1, attrR1, w16_1, b16_1, zrow,
                     conv1=True)
    agg1 = jnp.sum(agg1P[:27].reshape(3, 9, NP), axis=0)[:, :N].T

    x_pad = jnp.pad(x, ((0, 0), (0, 16 - D_IN)))
    agg1_pad = jnp.pad(agg1, ((0, 0), (0, 16 - D_IN)))
    w11p = jnp.pad(p["W11"], ((0, 16 - D_IN), (0, 0)))
    h1, s1, q1 = _dense_a(x_pad, agg1_pad, w11p, p["b11"].reshape(1, H))
    h2 = _dense_b(h1, s1, q1, p["g1"], p["beta1"], p["W12"],
                  p["b12"].reshape(1, H))

    sdR2, attrR2 = _edges_padded(sd, attr, E_PAD2)
    table2T = jnp.pad(h2.T, ((0, 0), (0, NP - N)))
    w16_2 = jnp.broadcast_to(p["We2"][0][:, None], (H, 16))
    b16_2 = jnp.broadcast_to(p["be2"][:, None], (H, 16))
    aggT = _conv_fm(table2T, sdR2, attrR2, w16_2, b16_2, zrow,
                    conv1=False)

    h3, s2, q2 = _dense_a(h2, aggT[:, :N].T, p["W21"], p["b21"].reshape(1, H))
    h4 = _dense_b(h3, s2, q2, p["g2"], p["beta2"], p["W22"],
                  p["b22"].reshape(1, H))

    batch3 = batch.reshape(NBLK, 1, NB)
    out = _pool_readout(h4, batch3, p["Wr1"], p["br1"].reshape(1, 32),
                        p["Wr2"], p["br2"].reshape(1, 1))
    return out


# speedup vs baseline: 1.3482x; 1.2425x over previous
import jax
import jax.numpy as jnp
from jax import lax
from jax.experimental import pallas as pl
from jax.experimental.pallas import tpu as pltpu
from jax.experimental.pallas import tpu_sc as plsc

N = 50000
E = 800000
D_IN = 9
H = 64
G = 512

NC = 2
NS = 16
C = 128
SUPER = 16

NB = 2000
NBLK = N // NB

NP = 50048

E_PAD2 = 802816
NR2 = E_PAD2 // C
NSUP2 = NR2 // SUPER

NSUP1R = 132
E_PAD1 = 3 * NSUP1R * SUPER * C
NR1 = E_PAD1 // C


def _conv_fm(tableT, sdR, attrR, w16, b16, zrow, conv1):
    mesh = plsc.VectorSubcoreMesh(core_axis_name="c", subcore_axis_name="s")
    out_rows = 32 if conv1 else H
    npass = 1 if conv1 else 2

    def body(tab_h, sd_h, attr_h, w_h, b_h, z_h, out_h,
             sdv, av, trow_v, acc_v, w_v, b_v, tsem, zsem, i0, i1):
        isems = [i0, i1]
        c = lax.axis_index("c")
        s = lax.axis_index("s")
        wid = c * NS + s
        if conv1:
            third = wid // 9
            active = third < 3
            start_sup = third * NSUP1R
            npairs = jnp.where(active, NSUP1R // 2, 0)
            nsup_t = NSUP1R
        else:
            active = wid >= 0
            start_sup = 0
            npairs = NSUP2 // 2
            nsup_t = NSUP2

        def stage(t, par):
            rb = (start_sup + t) * SUPER
            pb = par * SUPER
            pltpu.async_copy(sd_h.at[pl.ds(rb, SUPER)],
                             sdv.at[pl.ds(pb, SUPER)], isems[par])
            pltpu.async_copy(attr_h.at[pl.ds(rb, SUPER)],
                             av.at[pl.ds(pb, SUPER)], isems[par])

        def wait_stage(par):
            pb = par * SUPER
            pltpu.make_async_copy(sd_h.at[pl.ds(0, SUPER)],
                                  sdv.at[pl.ds(pb, SUPER)], isems[par]).wait()
            pltpu.make_async_copy(attr_h.at[pl.ds(0, SUPER)],
                                  av.at[pl.ds(pb, SUPER)], isems[par]).wait()

        for p in range(npass):
            if conv1:
                fid = wid % 9
                out_row = wid
            else:
                fid = p * 32 + wid
                out_row = fid
            pltpu.sync_copy(w_h.at[fid], w_v)
            pltpu.sync_copy(b_h.at[fid], b_v)
            wv = w_v[...]
            bv = b_v[...]
            tcopy = pltpu.async_copy(tab_h.at[fid], trow_v, tsem)
            zcopy = pltpu.async_copy(z_h, acc_v, zsem)

            @pl.when(active)
            def _prime():
                stage(0, 0)
            tcopy.wait()
            zcopy.wait()

            def sup2(t2, _):
                for par in (0, 1):
                    t = t2 * 2 + par

                    @pl.when(t + 1 < nsup_t)
                    def _next():
                        stage(t + 1, 1 - par)
                    wait_stage(par)

                    @plsc.parallel_loop(0, SUPER * (C // 16), unroll=2)
                    def _g(g):
                        row = par * SUPER + (g >> 3)
                        off = (g & 7) * 16
                        sd16 = sdv[row, pl.ds(off, 16)]
                        a16 = av[row, pl.ds(off, 16)]
                        s16 = sd16 & 0xFFFF
                        d16 = lax.shift_right_logical(sd16, 16)
                        vals = plsc.load_gather(trow_v, [s16])
                        m = jnp.maximum(vals + a16 * wv + bv, 0.0)
                        plsc.addupdate_scatter(acc_v, [d16], m)
                return 0
            lax.fori_loop(0, npairs, sup2, 0)
            pltpu.sync_copy(acc_v, out_h.at[out_row])

    return pl.kernel(
        body,
        out_type=jax.ShapeDtypeStruct((out_rows, NP), jnp.float32),
        mesh=mesh,
        compiler_params=pltpu.CompilerParams(use_tc_tiling_on_sc=False,
                                             needs_layout_passes=False),
        scratch_types=(
            [pltpu.VMEM((2 * SUPER, C), jnp.int32),
             pltpu.VMEM((2 * SUPER, C), jnp.float32),
             pltpu.VMEM((NP,), jnp.float32),
             pltpu.VMEM((NP,), jnp.float32),
             pltpu.VMEM((16,), jnp.float32),
             pltpu.VMEM((16,), jnp.float32)]
            + [pltpu.SemaphoreType.DMA] * 4
        ),
    )(tableT, sdR, attrR, w16, b16, zrow)


def _da_body(h_ref, a_ref, w_ref, b_ref, h1_ref, s_ref, q_ref):
    if a_ref.shape[0] == h_ref.shape[0]:
        h0 = h_ref[...] + a_ref[...]
    else:
        h0 = h_ref[...] + a_ref[...].T
    h1 = jnp.dot(h0, w_ref[...], preferred_element_type=jnp.float32) + b_ref[...]
    h1_ref[...] = h1
    s_ref[...] = jnp.sum(h1, axis=0, keepdims=True).reshape(1, 1, H)
    q_ref[...] = jnp.sum(h1 * h1, axis=0, keepdims=True).reshape(1, 1, H)


def _dense_a(h, agg, w, b, agg_t=False):
    k = w.shape[0]
    aspec = (pl.BlockSpec((k, NB), lambda i: (0, i)) if agg_t
             else pl.BlockSpec((NB, k), lambda i: (i, 0)))
    return pl.pallas_call(
        _da_body,
        grid=(NBLK,),
        in_specs=[
            pl.BlockSpec((NB, k), lambda i: (i, 0)),
            aspec,
            pl.BlockSpec((k, H), lambda i: (0, 0)),
            pl.BlockSpec((1, H), lambda i: (0, 0)),
        ],
        out_specs=[
            pl.BlockSpec((NB, H), lambda i: (i, 0)),
            pl.BlockSpec((1, 1, H), lambda i: (i, 0, 0)),
            pl.BlockSpec((1, 1, H), lambda i: (i, 0, 0)),
        ],
        out_shape=[
            jax.ShapeDtypeStruct((N, H), jnp.float32),
            jax.ShapeDtypeStruct((NBLK, 1, H), jnp.float32),
            jax.ShapeDtypeStruct((NBLK, 1, H), jnp.float32),
        ],
    )(h, agg, w, b)


def _db_body(h1_ref, s_ref, q_ref, g_ref, beta_ref, w_ref, b_ref, *out_refs):
    mean = jnp.sum(s_ref[...], axis=0) / N
    ex2 = jnp.sum(q_ref[...], axis=0) / N
    var = ex2 - mean * mean
    inv = lax.rsqrt(var + 1e-5)
    hb = g_ref[...] * (h1_ref[...] - mean) * inv + beta_ref[...]
    hr = jnp.maximum(hb, 0.0)
    o = jnp.dot(hr, w_ref[...], preferred_element_type=jnp.float32) + b_ref[...]
    o = jnp.maximum(o, 0.0)
    out_refs[0][...] = o
    if len(out_refs) > 1:
        out_refs[1][...] = o.T


def _dense_b(h1, s, q, g, beta, w, b, emit_t=False):
    out_specs = [pl.BlockSpec((NB, H), lambda i: (i, 0))]
    out_shape = [jax.ShapeDtypeStruct((N, H), jnp.float32)]
    if emit_t:
        out_specs.append(pl.BlockSpec((H, NB), lambda i: (0, i)))
        out_shape.append(jax.ShapeDtypeStruct((H, NP), jnp.float32))
    res = pl.pallas_call(
        _db_body,
        grid=(NBLK,),
        in_specs=[
            pl.BlockSpec((NB, H), lambda i: (i, 0)),
            pl.BlockSpec((NBLK, 1, H), lambda i: (0, 0, 0)),
            pl.BlockSpec((NBLK, 1, H), lambda i: (0, 0, 0)),
            pl.BlockSpec((1, H), lambda i: (0, 0)),
            pl.BlockSpec((1, H), lambda i: (0, 0)),
            pl.BlockSpec((H, H), lambda i: (0, 0)),
            pl.BlockSpec((1, H), lambda i: (0, 0)),
        ],
        out_specs=out_specs,
        out_shape=out_shape,
    )(h1, s, q, g.reshape(1, H), beta.reshape(1, H), w, b)
    return res if emit_t else res[0]


def _pool_body(h_ref, bt_ref, wr1_ref, br1_ref, wr2_ref, br2_ref, o_ref,
               acc_ref, cnt_ref):
    i = pl.program_id(0)

    @pl.when(i == 0)
    def _init():
        acc_ref[...] = jnp.zeros_like(acc_ref)
        cnt_ref[...] = jnp.zeros_like(cnt_ref)

    bt = bt_ref[0]
    gid = lax.broadcasted_iota(jnp.int32, (G, NB), 0)
    oh = (gid == bt).astype(jnp.float32)
    acc_ref[...] += jnp.dot(oh, h_ref[...], preferred_element_type=jnp.float32)
    cnt_part = jnp.sum(oh, axis=1, keepdims=True)
    cnt_ref[...] += jnp.broadcast_to(cnt_part, (G, H))

    @pl.when(i == NBLK - 1)
    def _final():
        xg = acc_ref[...] / jnp.maximum(cnt_ref[...], 1.0)
        r = jnp.maximum(
            jnp.dot(xg, wr1_ref[...], preferred_element_type=jnp.float32)
            + br1_ref[...], 0.0)
        o = jnp.dot(r, wr2_ref[...], preferred_element_type=jnp.float32) + br2_ref[...]
        o_ref[...] = 1.0 / (1.0 + jnp.exp(-o))


def _pool_readout(h4, batch3, wr1, br1, wr2, br2):
    return pl.pallas_call(
        _pool_body,
        grid=(NBLK,),
        in_specs=[
            pl.BlockSpec((NB, H), lambda i: (i, 0)),
            pl.BlockSpec((1, 1, NB), lambda i: (i, 0, 0)),
            pl.BlockSpec((H, 32), lambda i: (0, 0)),
            pl.BlockSpec((1, 32), lambda i: (0, 0)),
            pl.BlockSpec((32, 1), lambda i: (0, 0)),
            pl.BlockSpec((1, 1), lambda i: (0, 0)),
        ],
        out_specs=pl.BlockSpec((G, 1), lambda i: (0, 0)),
        out_shape=jax.ShapeDtypeStruct((G, 1), jnp.float32),
        scratch_shapes=[
            pltpu.VMEM((G, H), jnp.float32),
            pltpu.VMEM((G, H), jnp.float32),
        ],
    )(h4, batch3, wr1, br1, wr2, br2)


def _edges_padded(sd, attr, e_pad):
    pad = e_pad - E
    nr = e_pad // C
    sdR = jnp.concatenate([sd, jnp.full((pad,), N << 16, jnp.int32)]).reshape(nr, C)
    attrR = jnp.concatenate([attr, jnp.zeros((pad,), jnp.float32)]).reshape(nr, C)
    return sdR, attrR


def kernel(x, edge_index, edge_attr, batch, params):
    p = params
    f32 = jnp.float32
    src = edge_index[0]
    dst = edge_index[1]
    attr = edge_attr[:, 0]
    sd = src | (dst << 16)
    zrow = jnp.zeros((NP,), f32)

    sdR1, attrR1 = _edges_padded(sd, attr, E_PAD1)
    xT = jnp.pad(x.T, ((0, 0), (0, NP - N)))
    w16_1 = jnp.broadcast_to(p["We1"][0][:, None], (D_IN, 16))
    b16_1 = jnp.broadcast_to(p["be1"][:, None], (D_IN, 16))
    agg1P = _conv_fm(xT, sdR1, attrR1, w16_1, b16_1, zrow,
                     conv1=True)
    agg1 = jnp.sum(agg1P[:27].reshape(3, 9, NP), axis=0)[:, :N].T

    x_pad = jnp.pad(x, ((0, 0), (0, 16 - D_IN)))
    agg1_pad = jnp.pad(agg1, ((0, 0), (0, 16 - D_IN)))
    w11p = jnp.pad(p["W11"], ((0, 16 - D_IN), (0, 0)))
    h1, s1, q1 = _dense_a(x_pad, agg1_pad, w11p, p["b11"].reshape(1, H))
    h2 = _dense_b(h1, s1, q1, p["g1"], p["beta1"], p["W12"],
                  p["b12"].reshape(1, H))

    sdR2, attrR2 = _edges_padded(sd, attr, E_PAD2)
    table2T = jnp.pad(h2.T, ((0, 0), (0, NP - N)))
    w16_2 = jnp.broadcast_to(p["We2"][0][:, None], (H, 16))
    b16_2 = jnp.broadcast_to(p["be2"][:, None], (H, 16))
    aggT = _conv_fm(table2T, sdR2, attrR2, w16_2, b16_2, zrow,
                    conv1=False)

    h3, s2, q2 = _dense_a(h2, aggT[:, :N].T, p["W21"], p["b21"].reshape(1, H))
    h4 = _dense_b(h3, s2, q2, p["g2"], p["beta2"], p["W22"],
                  p["b22"].reshape(1, H))

    batch3 = batch.reshape(NBLK, 1, NB)
    out = _pool_readout(h4, batch3, p["Wr1"], p["br1"].reshape(1, 32),
                        p["Wr2"], p["br2"].reshape(1, 1))
    return out


# speedup vs baseline: 1.5567x; 1.1546x over previous
import jax
import jax.numpy as jnp
from jax import lax
from jax.experimental import pallas as pl
from jax.experimental.pallas import tpu as pltpu
from jax.experimental.pallas import tpu_sc as plsc

N = 50000
E = 800000
D_IN = 9
H = 64
G = 512

NC = 2
NS = 16
C = 128
SUPER = 32

NB = 2000
NBLK = N // NB

NP = 50048

E_PAD2 = 802816
NR2 = E_PAD2 // C
NSUP2 = NR2 // SUPER

NSUP1R = 66
E_PAD1 = 3 * NSUP1R * SUPER * C
NR1 = E_PAD1 // C


def _conv_fm(tableT, sdR, attrR, w16, b16, zrow, conv1):
    mesh = plsc.VectorSubcoreMesh(core_axis_name="c", subcore_axis_name="s")
    out_rows = 32 if conv1 else H
    npass = 1 if conv1 else 2

    def body(tab_h, sd_h, attr_h, w_h, b_h, z_h, out_h,
             sdv, av, trow_v, acc_v, w_v, b_v, tsem, zsem, i0, i1):
        isems = [i0, i1]
        c = lax.axis_index("c")
        s = lax.axis_index("s")
        wid = c * NS + s
        if conv1:
            third = wid // 9
            active = third < 3
            start_sup = third * NSUP1R
            npairs = jnp.where(active, NSUP1R // 2, 0)
            nsup_t = NSUP1R
        else:
            active = wid >= 0
            start_sup = 0
            npairs = NSUP2 // 2
            nsup_t = NSUP2

        def stage(t, par):
            rb = (start_sup + t) * SUPER
            pb = par * SUPER
            pltpu.async_copy(sd_h.at[pl.ds(rb, SUPER)],
                             sdv.at[pl.ds(pb, SUPER)], isems[par])
            pltpu.async_copy(attr_h.at[pl.ds(rb, SUPER)],
                             av.at[pl.ds(pb, SUPER)], isems[par])

        def wait_stage(par):
            pb = par * SUPER
            pltpu.make_async_copy(sd_h.at[pl.ds(0, SUPER)],
                                  sdv.at[pl.ds(pb, SUPER)], isems[par]).wait()
            pltpu.make_async_copy(attr_h.at[pl.ds(0, SUPER)],
                                  av.at[pl.ds(pb, SUPER)], isems[par]).wait()

        for p in range(npass):
            if conv1:
                fid = wid % 9
                out_row = wid
            else:
                fid = p * 32 + wid
                out_row = fid
            pltpu.sync_copy(w_h.at[fid], w_v)
            pltpu.sync_copy(b_h.at[fid], b_v)
            wv = w_v[...]
            bv = b_v[...]
            tcopy = pltpu.async_copy(tab_h.at[fid], trow_v, tsem)
            zcopy = pltpu.async_copy(z_h, acc_v, zsem)

            @pl.when(active)
            def _prime():
                stage(0, 0)
            tcopy.wait()
            zcopy.wait()

            def sup2(t2, _):
                for par in (0, 1):
                    t = t2 * 2 + par

                    @pl.when(t + 1 < nsup_t)
                    def _next():
                        stage(t + 1, 1 - par)
                    wait_stage(par)

                    @plsc.parallel_loop(0, SUPER * (C // 16), unroll=2)
                    def _g(g):
                        row = par * SUPER + (g >> 3)
                        off = (g & 7) * 16
                        sd16 = sdv[row, pl.ds(off, 16)]
                        a16 = av[row, pl.ds(off, 16)]
                        s16 = sd16 & 0xFFFF
                        d16 = lax.shift_right_logical(sd16, 16)
                        vals = plsc.load_gather(trow_v, [s16])
                        m = jnp.maximum(vals + a16 * wv + bv, 0.0)
                        plsc.addupdate_scatter(acc_v, [d16], m)
                return 0
            lax.fori_loop(0, npairs, sup2, 0)
            pltpu.sync_copy(acc_v, out_h.at[out_row])

    return pl.kernel(
        body,
        out_type=jax.ShapeDtypeStruct((out_rows, NP), jnp.float32),
        mesh=mesh,
        compiler_params=pltpu.CompilerParams(use_tc_tiling_on_sc=False,
                                             needs_layout_passes=False),
        scratch_types=(
            [pltpu.VMEM((2 * SUPER, C), jnp.int32),
             pltpu.VMEM((2 * SUPER, C), jnp.float32),
             pltpu.VMEM((NP,), jnp.float32),
             pltpu.VMEM((NP,), jnp.float32),
             pltpu.VMEM((16,), jnp.float32),
             pltpu.VMEM((16,), jnp.float32)]
            + [pltpu.SemaphoreType.DMA] * 4
        ),
    )(tableT, sdR, attrR, w16, b16, zrow)


def _da_body(h_ref, a_ref, w_ref, b_ref, h1_ref, s_ref, q_ref):
    if a_ref.shape[0] == h_ref.shape[0]:
        h0 = h_ref[...] + a_ref[...]
    else:
        h0 = h_ref[...] + a_ref[...].T
    h1 = jnp.dot(h0, w_ref[...], preferred_element_type=jnp.float32) + b_ref[...]
    h1_ref[...] = h1
    s_ref[...] = jnp.sum(h1, axis=0, keepdims=True).reshape(1, 1, H)
    q_ref[...] = jnp.sum(h1 * h1, axis=0, keepdims=True).reshape(1, 1, H)


def _dense_a(h, agg, w, b, agg_t=False):
    k = w.shape[0]
    aspec = (pl.BlockSpec((k, NB), lambda i: (0, i)) if agg_t
             else pl.BlockSpec((NB, k), lambda i: (i, 0)))
    return pl.pallas_call(
        _da_body,
        grid=(NBLK,),
        in_specs=[
            pl.BlockSpec((NB, k), lambda i: (i, 0)),
            aspec,
            pl.BlockSpec((k, H), lambda i: (0, 0)),
            pl.BlockSpec((1, H), lambda i: (0, 0)),
        ],
        out_specs=[
            pl.BlockSpec((NB, H), lambda i: (i, 0)),
            pl.BlockSpec((1, 1, H), lambda i: (i, 0, 0)),
            pl.BlockSpec((1, 1, H), lambda i: (i, 0, 0)),
        ],
        out_shape=[
            jax.ShapeDtypeStruct((N, H), jnp.float32),
            jax.ShapeDtypeStruct((NBLK, 1, H), jnp.float32),
            jax.ShapeDtypeStruct((NBLK, 1, H), jnp.float32),
        ],
    )(h, agg, w, b)


def _db_body(h1_ref, s_ref, q_ref, g_ref, beta_ref, w_ref, b_ref, *out_refs):
    mean = jnp.sum(s_ref[...], axis=0) / N
    ex2 = jnp.sum(q_ref[...], axis=0) / N
    var = ex2 - mean * mean
    inv = lax.rsqrt(var + 1e-5)
    hb = g_ref[...] * (h1_ref[...] - mean) * inv + beta_ref[...]
    hr = jnp.maximum(hb, 0.0)
    o = jnp.dot(hr, w_ref[...], preferred_element_type=jnp.float32) + b_ref[...]
    o = jnp.maximum(o, 0.0)
    out_refs[0][...] = o
    if len(out_refs) > 1:
        out_refs[1][...] = o.T


def _dense_b(h1, s, q, g, beta, w, b, emit_t=False):
    out_specs = [pl.BlockSpec((NB, H), lambda i: (i, 0))]
    out_shape = [jax.ShapeDtypeStruct((N, H), jnp.float32)]
    if emit_t:
        out_specs.append(pl.BlockSpec((H, NB), lambda i: (0, i)))
        out_shape.append(jax.ShapeDtypeStruct((H, NP), jnp.float32))
    res = pl.pallas_call(
        _db_body,
        grid=(NBLK,),
        in_specs=[
            pl.BlockSpec((NB, H), lambda i: (i, 0)),
            pl.BlockSpec((NBLK, 1, H), lambda i: (0, 0, 0)),
            pl.BlockSpec((NBLK, 1, H), lambda i: (0, 0, 0)),
            pl.BlockSpec((1, H), lambda i: (0, 0)),
            pl.BlockSpec((1, H), lambda i: (0, 0)),
            pl.BlockSpec((H, H), lambda i: (0, 0)),
            pl.BlockSpec((1, H), lambda i: (0, 0)),
        ],
        out_specs=out_specs,
        out_shape=out_shape,
    )(h1, s, q, g.reshape(1, H), beta.reshape(1, H), w, b)
    return res if emit_t else res[0]


def _pool_body(h_ref, bt_ref, wr1_ref, br1_ref, wr2_ref, br2_ref, o_ref,
               acc_ref, cnt_ref):
    i = pl.program_id(0)

    @pl.when(i == 0)
    def _init():
        acc_ref[...] = jnp.zeros_like(acc_ref)
        cnt_ref[...] = jnp.zeros_like(cnt_ref)

    bt = bt_ref[0]
    gid = lax.broadcasted_iota(jnp.int32, (G, NB), 0)
    oh = (gid == bt).astype(jnp.float32)
    acc_ref[...] += jnp.dot(oh, h_ref[...], preferred_element_type=jnp.float32)
    cnt_part = jnp.sum(oh, axis=1, keepdims=True)
    cnt_ref[...] += jnp.broadcast_to(cnt_part, (G, H))

    @pl.when(i == NBLK - 1)
    def _final():
        xg = acc_ref[...] / jnp.maximum(cnt_ref[...], 1.0)
        r = jnp.maximum(
            jnp.dot(xg, wr1_ref[...], preferred_element_type=jnp.float32)
            + br1_ref[...], 0.0)
        o = jnp.dot(r, wr2_ref[...], preferred_element_type=jnp.float32) + br2_ref[...]
        o_ref[...] = 1.0 / (1.0 + jnp.exp(-o))


def _pool_readout(h4, batch3, wr1, br1, wr2, br2):
    return pl.pallas_call(
        _pool_body,
        grid=(NBLK,),
        in_specs=[
            pl.BlockSpec((NB, H), lambda i: (i, 0)),
            pl.BlockSpec((1, 1, NB), lambda i: (i, 0, 0)),
            pl.BlockSpec((H, 32), lambda i: (0, 0)),
            pl.BlockSpec((1, 32), lambda i: (0, 0)),
            pl.BlockSpec((32, 1), lambda i: (0, 0)),
            pl.BlockSpec((1, 1), lambda i: (0, 0)),
        ],
        out_specs=pl.BlockSpec((G, 1), lambda i: (0, 0)),
        out_shape=jax.ShapeDtypeStruct((G, 1), jnp.float32),
        scratch_shapes=[
            pltpu.VMEM((G, H), jnp.float32),
            pltpu.VMEM((G, H), jnp.float32),
        ],
    )(h4, batch3, wr1, br1, wr2, br2)


def _edges_padded(sd, attr, e_pad):
    pad = e_pad - E
    nr = e_pad // C
    sdR = jnp.concatenate([sd, jnp.full((pad,), N << 16, jnp.int32)]).reshape(nr, C)
    attrR = jnp.concatenate([attr, jnp.zeros((pad,), jnp.float32)]).reshape(nr, C)
    return sdR, attrR


def kernel(x, edge_index, edge_attr, batch, params):
    p = params
    f32 = jnp.float32
    src = edge_index[0]
    dst = edge_index[1]
    attr = edge_attr[:, 0]
    sd = src | (dst << 16)
    zrow = jnp.zeros((NP,), f32)

    sdR1, attrR1 = _edges_padded(sd, attr, E_PAD1)
    xT = jnp.pad(x.T, ((0, 0), (0, NP - N)))
    w16_1 = jnp.broadcast_to(p["We1"][0][:, None], (D_IN, 16))
    b16_1 = jnp.broadcast_to(p["be1"][:, None], (D_IN, 16))
    agg1P = _conv_fm(xT, sdR1, attrR1, w16_1, b16_1, zrow,
                     conv1=True)
    agg1 = jnp.sum(agg1P[:27].reshape(3, 9, NP), axis=0)[:, :N].T

    x_pad = jnp.pad(x, ((0, 0), (0, 16 - D_IN)))
    agg1_pad = jnp.pad(agg1, ((0, 0), (0, 16 - D_IN)))
    w11p = jnp.pad(p["W11"], ((0, 16 - D_IN), (0, 0)))
    h1, s1, q1 = _dense_a(x_pad, agg1_pad, w11p, p["b11"].reshape(1, H))
    h2 = _dense_b(h1, s1, q1, p["g1"], p["beta1"], p["W12"],
                  p["b12"].reshape(1, H))

    sdR2, attrR2 = _edges_padded(sd, attr, E_PAD2)
    table2T = jnp.pad(h2.T, ((0, 0), (0, NP - N)))
    w16_2 = jnp.broadcast_to(p["We2"][0][:, None], (H, 16))
    b16_2 = jnp.broadcast_to(p["be2"][:, None], (H, 16))
    aggT = _conv_fm(table2T, sdR2, attrR2, w16_2, b16_2, zrow,
                    conv1=False)

    h3, s2, q2 = _dense_a(h2, aggT[:, :N].T, p["W21"], p["b21"].reshape(1, H))
    h4 = _dense_b(h3, s2, q2, p["g2"], p["beta2"], p["W22"],
                  p["b22"].reshape(1, H))

    batch3 = batch.reshape(NBLK, 1, NB)
    out = _pool_readout(h4, batch3, p["Wr1"], p["br1"].reshape(1, 32),
                        p["Wr2"], p["br2"].reshape(1, 1))
    return out


# speedup vs baseline: 1.5901x; 1.0215x over previous
import jax
import jax.numpy as jnp
from jax import lax
from jax.experimental import pallas as pl
from jax.experimental.pallas import tpu as pltpu
from jax.experimental.pallas import tpu_sc as plsc

N = 50000
E = 800000
D_IN = 9
H = 64
G = 512

NC = 2
NS = 16
C = 128
SUPER = 48

NB = 2000
NBLK = N // NB

NP = 50048

E_PAD2 = 811008
NR2 = E_PAD2 // C
NSUP2 = NR2 // SUPER

NSUP1R = 44
E_PAD1 = 3 * NSUP1R * SUPER * C
NR1 = E_PAD1 // C


def _conv_fm(tableT, sdR, attrR, w16, b16, zrow, conv1):
    mesh = plsc.VectorSubcoreMesh(core_axis_name="c", subcore_axis_name="s")
    out_rows = 32 if conv1 else H
    npass = 1 if conv1 else 2

    def body(tab_h, sd_h, attr_h, w_h, b_h, z_h, out_h,
             sdv, av, trow_v, acc_v, w_v, b_v, tsem, zsem, i0, i1):
        isems = [i0, i1]
        c = lax.axis_index("c")
        s = lax.axis_index("s")
        wid = c * NS + s
        if conv1:
            third = wid // 9
            active = third < 3
            start_sup = third * NSUP1R
            npairs = jnp.where(active, NSUP1R // 2, 0)
            nsup_t = NSUP1R
        else:
            active = wid >= 0
            start_sup = 0
            npairs = NSUP2 // 2
            nsup_t = NSUP2

        def stage(t, par):
            rb = (start_sup + t) * SUPER
            pb = par * SUPER
            pltpu.async_copy(sd_h.at[pl.ds(rb, SUPER)],
                             sdv.at[pl.ds(pb, SUPER)], isems[par])
            pltpu.async_copy(attr_h.at[pl.ds(rb, SUPER)],
                             av.at[pl.ds(pb, SUPER)], isems[par])

        def wait_stage(par):
            pb = par * SUPER
            pltpu.make_async_copy(sd_h.at[pl.ds(0, SUPER)],
                                  sdv.at[pl.ds(pb, SUPER)], isems[par]).wait()
            pltpu.make_async_copy(attr_h.at[pl.ds(0, SUPER)],
                                  av.at[pl.ds(pb, SUPER)], isems[par]).wait()

        for p in range(npass):
            if conv1:
                fid = wid % 9
                out_row = wid
            else:
                fid = p * 32 + wid
                out_row = fid
            pltpu.sync_copy(w_h.at[fid], w_v)
            pltpu.sync_copy(b_h.at[fid], b_v)
            wv = w_v[...]
            bv = b_v[...]
            tcopy = pltpu.async_copy(tab_h.at[fid], trow_v, tsem)
            zcopy = pltpu.async_copy(z_h, acc_v, zsem)

            @pl.when(active)
            def _prime():
                stage(0, 0)
            tcopy.wait()
            zcopy.wait()

            def sup2(t2, _):
                for par in (0, 1):
                    t = t2 * 2 + par

                    @pl.when(t + 1 < nsup_t)
                    def _next():
                        stage(t + 1, 1 - par)
                    wait_stage(par)

                    @plsc.parallel_loop(0, SUPER * (C // 16), unroll=2)
                    def _g(g):
                        row = par * SUPER + (g >> 3)
                        off = (g & 7) * 16
                        sd16 = sdv[row, pl.ds(off, 16)]
                        a16 = av[row, pl.ds(off, 16)]
                        s16 = sd16 & 0xFFFF
                        d16 = lax.shift_right_logical(sd16, 16)
                        vals = plsc.load_gather(trow_v, [s16])
                        m = jnp.maximum(vals + a16 * wv + bv, 0.0)
                        plsc.addupdate_scatter(acc_v, [d16], m)
                return 0
            lax.fori_loop(0, npairs, sup2, 0)
            pltpu.sync_copy(acc_v, out_h.at[out_row])

    return pl.kernel(
        body,
        out_type=jax.ShapeDtypeStruct((out_rows, NP), jnp.float32),
        mesh=mesh,
        compiler_params=pltpu.CompilerParams(use_tc_tiling_on_sc=False,
                                             needs_layout_passes=False),
        scratch_types=(
            [pltpu.VMEM((2 * SUPER, C), jnp.int32),
             pltpu.VMEM((2 * SUPER, C), jnp.float32),
             pltpu.VMEM((NP,), jnp.float32),
             pltpu.VMEM((NP,), jnp.float32),
             pltpu.VMEM((16,), jnp.float32),
             pltpu.VMEM((16,), jnp.float32)]
            + [pltpu.SemaphoreType.DMA] * 4
        ),
    )(tableT, sdR, attrR, w16, b16, zrow)


def _da_body(h_ref, a_ref, w_ref, b_ref, h1_ref, s_ref, q_ref):
    if a_ref.shape[0] == h_ref.shape[0]:
        h0 = h_ref[...] + a_ref[...]
    else:
        h0 = h_ref[...] + a_ref[...].T
    h1 = jnp.dot(h0, w_ref[...], preferred_element_type=jnp.float32) + b_ref[...]
    h1_ref[...] = h1
    s_ref[...] = jnp.sum(h1, axis=0, keepdims=True).reshape(1, 1, H)
    q_ref[...] = jnp.sum(h1 * h1, axis=0, keepdims=True).reshape(1, 1, H)


def _dense_a(h, agg, w, b, agg_t=False):
    k = w.shape[0]
    aspec = (pl.BlockSpec((k, NB), lambda i: (0, i)) if agg_t
             else pl.BlockSpec((NB, k), lambda i: (i, 0)))
    return pl.pallas_call(
        _da_body,
        grid=(NBLK,),
        in_specs=[
            pl.BlockSpec((NB, k), lambda i: (i, 0)),
            aspec,
            pl.BlockSpec((k, H), lambda i: (0, 0)),
            pl.BlockSpec((1, H), lambda i: (0, 0)),
        ],
        out_specs=[
            pl.BlockSpec((NB, H), lambda i: (i, 0)),
            pl.BlockSpec((1, 1, H), lambda i: (i, 0, 0)),
            pl.BlockSpec((1, 1, H), lambda i: (i, 0, 0)),
        ],
        out_shape=[
            jax.ShapeDtypeStruct((N, H), jnp.float32),
            jax.ShapeDtypeStruct((NBLK, 1, H), jnp.float32),
            jax.ShapeDtypeStruct((NBLK, 1, H), jnp.float32),
        ],
    )(h, agg, w, b)


def _db_body(h1_ref, s_ref, q_ref, g_ref, beta_ref, w_ref, b_ref, *out_refs):
    mean = jnp.sum(s_ref[...], axis=0) / N
    ex2 = jnp.sum(q_ref[...], axis=0) / N
    var = ex2 - mean * mean
    inv = lax.rsqrt(var + 1e-5)
    hb = g_ref[...] * (h1_ref[...] - mean) * inv + beta_ref[...]
    hr = jnp.maximum(hb, 0.0)
    o = jnp.dot(hr, w_ref[...], preferred_element_type=jnp.float32) + b_ref[...]
    o = jnp.maximum(o, 0.0)
    out_refs[0][...] = o
    if len(out_refs) > 1:
        out_refs[1][...] = o.T


def _dense_b(h1, s, q, g, beta, w, b, emit_t=False):
    out_specs = [pl.BlockSpec((NB, H), lambda i: (i, 0))]
    out_shape = [jax.ShapeDtypeStruct((N, H), jnp.float32)]
    if emit_t:
        out_specs.append(pl.BlockSpec((H, NB), lambda i: (0, i)))
        out_shape.append(jax.ShapeDtypeStruct((H, NP), jnp.float32))
    res = pl.pallas_call(
        _db_body,
        grid=(NBLK,),
        in_specs=[
            pl.BlockSpec((NB, H), lambda i: (i, 0)),
            pl.BlockSpec((NBLK, 1, H), lambda i: (0, 0, 0)),
            pl.BlockSpec((NBLK, 1, H), lambda i: (0, 0, 0)),
            pl.BlockSpec((1, H), lambda i: (0, 0)),
            pl.BlockSpec((1, H), lambda i: (0, 0)),
            pl.BlockSpec((H, H), lambda i: (0, 0)),
            pl.BlockSpec((1, H), lambda i: (0, 0)),
        ],
        out_specs=out_specs,
        out_shape=out_shape,
    )(h1, s, q, g.reshape(1, H), beta.reshape(1, H), w, b)
    return res if emit_t else res[0]


def _pool_body(h_ref, bt_ref, wr1_ref, br1_ref, wr2_ref, br2_ref, o_ref,
               acc_ref, cnt_ref):
    i = pl.program_id(0)

    @pl.when(i == 0)
    def _init():
        acc_ref[...] = jnp.zeros_like(acc_ref)
        cnt_ref[...] = jnp.zeros_like(cnt_ref)

    bt = bt_ref[0]
    gid = lax.broadcasted_iota(jnp.int32, (G, NB), 0)
    oh = (gid == bt).astype(jnp.float32)
    acc_ref[...] += jnp.dot(oh, h_ref[...], preferred_element_type=jnp.float32)
    cnt_part = jnp.sum(oh, axis=1, keepdims=True)
    cnt_ref[...] += jnp.broadcast_to(cnt_part, (G, H))

    @pl.when(i == NBLK - 1)
    def _final():
        xg = acc_ref[...] / jnp.maximum(cnt_ref[...], 1.0)
        r = jnp.maximum(
            jnp.dot(xg, wr1_ref[...], preferred_element_type=jnp.float32)
            + br1_ref[...], 0.0)
        o = jnp.dot(r, wr2_ref[...], preferred_element_type=jnp.float32) + br2_ref[...]
        o_ref[...] = 1.0 / (1.0 + jnp.exp(-o))


def _pool_readout(h4, batch3, wr1, br1, wr2, br2):
    return pl.pallas_call(
        _pool_body,
        grid=(NBLK,),
        in_specs=[
            pl.BlockSpec((NB, H), lambda i: (i, 0)),
            pl.BlockSpec((1, 1, NB), lambda i: (i, 0, 0)),
            pl.BlockSpec((H, 32), lambda i: (0, 0)),
            pl.BlockSpec((1, 32), lambda i: (0, 0)),
            pl.BlockSpec((32, 1), lambda i: (0, 0)),
            pl.BlockSpec((1, 1), lambda i: (0, 0)),
        ],
        out_specs=pl.BlockSpec((G, 1), lambda i: (0, 0)),
        out_shape=jax.ShapeDtypeStruct((G, 1), jnp.float32),
        scratch_shapes=[
            pltpu.VMEM((G, H), jnp.float32),
            pltpu.VMEM((G, H), jnp.float32),
        ],
    )(h4, batch3, wr1, br1, wr2, br2)


def _edges_padded(sd, attr, e_pad):
    pad = e_pad - E
    nr = e_pad // C
    sdR = jnp.concatenate([sd, jnp.full((pad,), N << 16, jnp.int32)]).reshape(nr, C)
    attrR = jnp.concatenate([attr, jnp.zeros((pad,), jnp.float32)]).reshape(nr, C)
    return sdR, attrR


def kernel(x, edge_index, edge_attr, batch, params):
    p = params
    f32 = jnp.float32
    src = edge_index[0]
    dst = edge_index[1]
    attr = edge_attr[:, 0]
    sd = src | (dst << 16)
    zrow = jnp.zeros((NP,), f32)

    sdR1, attrR1 = _edges_padded(sd, attr, E_PAD1)
    xT = jnp.pad(x.T, ((0, 0), (0, NP - N)))
    w16_1 = jnp.broadcast_to(p["We1"][0][:, None], (D_IN, 16))
    b16_1 = jnp.broadcast_to(p["be1"][:, None], (D_IN, 16))
    agg1P = _conv_fm(xT, sdR1, attrR1, w16_1, b16_1, zrow,
                     conv1=True)
    agg1 = jnp.sum(agg1P[:27].reshape(3, 9, NP), axis=0)[:, :N].T

    x_pad = jnp.pad(x, ((0, 0), (0, 16 - D_IN)))
    agg1_pad = jnp.pad(agg1, ((0, 0), (0, 16 - D_IN)))
    w11p = jnp.pad(p["W11"], ((0, 16 - D_IN), (0, 0)))
    h1, s1, q1 = _dense_a(x_pad, agg1_pad, w11p, p["b11"].reshape(1, H))
    h2 = _dense_b(h1, s1, q1, p["g1"], p["beta1"], p["W12"],
                  p["b12"].reshape(1, H))

    sdR2, attrR2 = _edges_padded(sd, attr, E_PAD2)
    table2T = jnp.pad(h2.T, ((0, 0), (0, NP - N)))
    w16_2 = jnp.broadcast_to(p["We2"][0][:, None], (H, 16))
    b16_2 = jnp.broadcast_to(p["be2"][:, None], (H, 16))
    aggT = _conv_fm(table2T, sdR2, attrR2, w16_2, b16_2, zrow,
                    conv1=False)

    h3, s2, q2 = _dense_a(h2, aggT[:, :N].T, p["W21"], p["b21"].reshape(1, H))
    h4 = _dense_b(h3, s2, q2, p["g2"], p["beta2"], p["W22"],
                  p["b22"].reshape(1, H))

    batch3 = batch.reshape(NBLK, 1, NB)
    out = _pool_readout(h4, batch3, p["Wr1"], p["br1"].reshape(1, 32),
                        p["Wr2"], p["br2"].reshape(1, 1))
    return out
